# baseline (speedup 1.0000x reference)
import functools

import jax
import jax.numpy as jnp
from jax import lax
from jax.experimental import pallas as pl
from jax.experimental.pallas import tpu as pltpu
from jax.experimental.pallas import tpu_sc as plsc

NC = 2
NS = 16
NW = NC * NS
L = 16
D = 64
B = 16384
BPW = B // NW
CH = 128
NCHUNK = BPW // CH

_MESH = plsc.VectorSubcoreMesh(
    core_axis_name="c", subcore_axis_name="s", num_cores=NC, num_subcores=NS
)


def _neg_sqrt(x):
    xs = jnp.maximum(x, jnp.float32(1e-30))
    i = lax.bitcast_convert_type(xs, jnp.int32)
    y = lax.bitcast_convert_type(
        jnp.int32(0x5F3759DF) - lax.shift_right_logical(i, 1), jnp.float32
    )
    half = jnp.float32(0.5) * xs
    for _ in range(3):
        y = y * (jnp.float32(1.5) - half * y * y)
    return -(xs * y)


def _body(
    heads_hbm, tails_hbm, rels_hbm, nheads_hbm, ntails_hbm, ent_hbm, rel_hbm,
    pos_hbm, neg_hbm,
    hi_v, ti_v, ri_v, nhi_v, nti_v,
    hrow_v, trow_v, rrow_v, nhrow_v, ntrow_v,
    pp_v, nn_v, pos_v, neg_v, sem,
):
    wid = lax.axis_index("s") * NC + lax.axis_index("c")
    base = wid * BPW

    def chunk_body(c, _):
        off = base + c * CH
        pltpu.sync_copy(heads_hbm.at[pl.ds(off, CH)], hi_v)
        pltpu.sync_copy(tails_hbm.at[pl.ds(off, CH)], ti_v)
        pltpu.sync_copy(rels_hbm.at[pl.ds(off, CH)], ri_v)
        pltpu.sync_copy(nheads_hbm.at[pl.ds(off, CH)], nhi_v)
        pltpu.sync_copy(ntails_hbm.at[pl.ds(off, CH)], nti_v)
        cps = [
            pltpu.async_copy(ent_hbm.at[hi_v], hrow_v, sem),
            pltpu.async_copy(ent_hbm.at[ti_v], trow_v, sem),
            pltpu.async_copy(rel_hbm.at[ri_v], rrow_v, sem),
            pltpu.async_copy(ent_hbm.at[nhi_v], nhrow_v, sem),
            pltpu.async_copy(ent_hbm.at[nti_v], ntrow_v, sem),
        ]
        for cp in cps:
            cp.wait()

        def row_body(r, _):
            accp = jnp.zeros((L,), jnp.float32)
            accn = jnp.zeros((L,), jnp.float32)
            for j in range(D // L):
                sl = pl.ds(j * L, L)
                rr = rrow_v[r, sl]
                dp = hrow_v[r, sl] + rr - trow_v[r, sl]
                dn = nhrow_v[r, sl] + rr - ntrow_v[r, sl]
                accp = accp + dp * dp
                accn = accn + dn * dn
            pp_v[pl.ds(r * L, L)] = accp
            nn_v[pl.ds(r * L, L)] = accn
            return 0

        lax.fori_loop(0, CH, row_body, 0, unroll=2)

        def grp_body(g, _):
            rows = (g * L + lax.iota(jnp.int32, L)) * L
            sp = jnp.zeros((L,), jnp.float32)
            sn = jnp.zeros((L,), jnp.float32)
            for j in range(L):
                idx = rows + j
                sp = sp + plsc.load_gather(pp_v, [idx])
                sn = sn + plsc.load_gather(nn_v, [idx])
            sl = pl.ds(g * L, L)
            pos_v[sl] = _neg_sqrt(sp)
            neg_v[sl] = _neg_sqrt(sn)
            return 0

        lax.fori_loop(0, CH // L, grp_body, 0)

        pltpu.sync_copy(pos_v, pos_hbm.at[pl.ds(off, CH)])
        pltpu.sync_copy(neg_v, neg_hbm.at[pl.ds(off, CH)])
        return 0

    lax.fori_loop(0, NCHUNK, chunk_body, 0)


_sc_call = pl.kernel(
    _body,
    out_type=(
        jax.ShapeDtypeStruct((B,), jnp.float32),
        jax.ShapeDtypeStruct((B,), jnp.float32),
    ),
    mesh=_MESH,
    scratch_types=[
        pltpu.VMEM((CH,), jnp.int32),
        pltpu.VMEM((CH,), jnp.int32),
        pltpu.VMEM((CH,), jnp.int32),
        pltpu.VMEM((CH,), jnp.int32),
        pltpu.VMEM((CH,), jnp.int32),
        pltpu.VMEM((CH, D), jnp.float32),
        pltpu.VMEM((CH, D), jnp.float32),
        pltpu.VMEM((CH, D), jnp.float32),
        pltpu.VMEM((CH, D), jnp.float32),
        pltpu.VMEM((CH, D), jnp.float32),
        pltpu.VMEM((CH * L,), jnp.float32),
        pltpu.VMEM((CH * L,), jnp.float32),
        pltpu.VMEM((CH,), jnp.float32),
        pltpu.VMEM((CH,), jnp.float32),
        pltpu.SemaphoreType.DMA,
    ],
    compiler_params=pltpu.CompilerParams(
        needs_layout_passes=False, use_tc_tiling_on_sc=False
    ),
    name="transe_score_sc",
)


def kernel(heads, tails, relations, negative_heads, negative_tails, ent_emb, rel_emb):
    heads = heads.astype(jnp.int32)
    tails = tails.astype(jnp.int32)
    relations = relations.astype(jnp.int32)
    negative_heads = negative_heads.astype(jnp.int32)
    negative_tails = negative_tails.astype(jnp.int32)
    pos, neg = _sc_call(
        heads, tails, relations, negative_heads, negative_tails, ent_emb, rel_emb
    )
    return (pos, neg)
